# Initial kernel scaffold; baseline (speedup 1.0000x reference)
import jax
import jax.numpy as jnp
from jax.experimental import pallas as pl


def kernel(candidate_pts, src_keypts, tgt_pts_xyz, tgt_deep_feat_pts):
    raise NotImplementedError("write your pallas kernel here")


# speedup vs baseline: 11.1384x; 11.1384x over previous
import functools

import jax
import jax.numpy as jnp
from jax import lax
from jax.experimental import pallas as pl
from jax.experimental.pallas import tpu as pltpu
from jax.experimental.pallas import tpu_sc as plsc

NSAMPLE = 32
K_NN = 32
RADIUS = 1.0
NQ = 1024
NT = 50000
CF = 32


def _square_distance(src, dst):
    return (jnp.sum(src ** 2, -1)[:, :, None]
            + jnp.sum(dst ** 2, -1)[:, None, :]
            - 2.0 * jnp.einsum('bsc,bnc->bsn', src, dst))


def _fps_ball_body(cand_ref, cand_rows_ref, sqd_ref, fps_ref, g_ref):
    x = cand_rows_ref[0:1, :]
    y = cand_rows_ref[1:2, :]
    z = cand_rows_ref[2:3, :]
    iota_row = lax.broadcasted_iota(jnp.int32, (1, NQ), 1)

    def fps_step(i, state):
        cent, distance, farthest = state
        cent = jnp.where(iota_row == i, farthest, cent)
        sel = iota_row == farthest
        cx = jnp.sum(jnp.where(sel, x, 0.0))
        cy = jnp.sum(jnp.where(sel, y, 0.0))
        cz = jnp.sum(jnp.where(sel, z, 0.0))
        d = (x - cx) ** 2 + (y - cy) ** 2 + (z - cz) ** 2
        distance = jnp.minimum(distance, d)
        m = jnp.max(distance)
        cand_idx = jnp.where(distance == m, iota_row, NQ)
        farthest = jnp.min(cand_idx)
        return cent, distance, farthest

    init = (jnp.zeros((1, NQ), jnp.int32),
            jnp.full((1, NQ), 1e10, jnp.float32),
            jnp.zeros((), jnp.int32))
    cent, _, _ = lax.fori_loop(0, NQ, fps_step, init)
    fps_ref[...] = cent

    sqd = sqd_ref[...]
    mask = (sqd <= RADIUS * RADIUS).astype(jnp.float32)
    iota0 = lax.broadcasted_iota(jnp.int32, (NQ, NQ), 0)
    iota1 = lax.broadcasted_iota(jnp.int32, (NQ, NQ), 1)
    tri = (iota0 <= iota1).astype(jnp.float32)
    cnt = jnp.dot(mask, tri, preferred_element_type=jnp.float32)
    total = cnt[:, NQ - 1:NQ]

    g_ref[:, 3 * NSAMPLE:] = jnp.zeros((NQ, 128 - 3 * NSAMPLE), jnp.float32)
    xr = x
    yr = y
    zr = z
    xq = cand_ref[:, 0:1]
    yq = cand_ref[:, 1:2]
    zq = cand_ref[:, 2:3]

    g0x = g0y = g0z = None
    for s in range(NSAMPLE):
        sel = jnp.logical_and(cnt == float(s + 1), mask > 0.0).astype(jnp.float32)
        gx = jnp.sum(sel * xr, axis=1, keepdims=True)
        gy = jnp.sum(sel * yr, axis=1, keepdims=True)
        gz = jnp.sum(sel * zr, axis=1, keepdims=True)
        if s == 0:
            g0x, g0y, g0z = gx, gy, gz
        else:
            valid = total >= float(s + 1)
            gx = jnp.where(valid, gx, g0x)
            gy = jnp.where(valid, gy, g0y)
            gz = jnp.where(valid, gz, g0z)
        g_ref[:, 3 * s + 0:3 * s + 1] = gx - xq
        g_ref[:, 3 * s + 1:3 * s + 2] = gy - yq
        g_ref[:, 3 * s + 2:3 * s + 3] = gz - zq


def _run_fps_ball(cand, cand_rows, sqd):
    return pl.pallas_call(
        _fps_ball_body,
        out_shape=(jax.ShapeDtypeStruct((1, NQ), jnp.int32),
                   jax.ShapeDtypeStruct((NQ, 128), jnp.float32)),
    )(cand, cand_rows, sqd)


_QB = 128
_NT_PAD = 50176
_WC = 6272
_NC = _NT_PAD // _WC
_BIG = 3e38


def _topk_body(sq_ref, idx_ref, e_ref, d_scr, bd_scr, bi_scr):
    ch = pl.program_id(1)

    @pl.when(ch == 0)
    def _init():
        bd_scr[...] = jnp.full((_QB, K_NN), _BIG, jnp.float32)
        bi_scr[...] = jnp.zeros((_QB, K_NN), jnp.int32)

    d_scr[...] = jnp.sqrt(jnp.maximum(sq_ref[...], 1e-12))
    iota_c = lax.broadcasted_iota(jnp.int32, (_QB, _WC), 1) + ch * _WC
    kiota = lax.broadcasted_iota(jnp.int32, (_QB, K_NN), 1)

    for _ in range(K_NN):
        d = d_scr[...]
        m = jnp.min(d, axis=1, keepdims=True)
        im = jnp.min(jnp.where(d == m, iota_c, jnp.int32(2 ** 30)),
                     axis=1, keepdims=True)
        d_scr[...] = jnp.where(iota_c == im, _BIG, d)
        bd = bd_scr[...]
        wm = jnp.max(bd, axis=1, keepdims=True)
        ins = m < wm
        pos = jnp.min(jnp.where(bd == wm, kiota, jnp.int32(2 ** 30)),
                      axis=1, keepdims=True)
        repl = jnp.logical_and(ins, kiota == pos)
        bd_scr[...] = jnp.where(repl, m, bd)
        bi_scr[...] = jnp.where(repl, im, bi_scr[...])

    @pl.when(ch == _NC - 1)
    def _finish():
        bd = bd_scr[...]
        bi = bi_scr[...]
        ms = []
        for j in range(K_NN):
            m = jnp.min(bd, axis=1, keepdims=True)
            im = jnp.min(jnp.where(bd == m, bi, jnp.int32(2 ** 30)),
                         axis=1, keepdims=True)
            idx_ref[:, j:j + 1] = im
            ms.append(m)
            bd = jnp.where(jnp.logical_and(bd == m, bi == im), _BIG, bd)
        dvals = jnp.concatenate(ms, axis=1)
        dsum = jnp.sum(dvals, axis=1, keepdims=True)
        w = dvals / dsum
        for k in range(K_NN):
            e_ref[:, CF * k:CF * (k + 1)] = jnp.broadcast_to(
                w[:, k:k + 1], (_QB, CF))


def _run_topk(sq):
    grid = (NQ // _QB, _NC)
    sq = jnp.pad(sq, ((0, 0), (0, _NT_PAD - NT)), constant_values=_BIG)
    return pl.pallas_call(
        _topk_body,
        grid=grid,
        in_specs=[pl.BlockSpec((_QB, _WC), lambda i, j: (i, j))],
        out_specs=(pl.BlockSpec((_QB, K_NN), lambda i, j: (i, 0)),
                   pl.BlockSpec((_QB, K_NN * CF), lambda i, j: (i, 0))),
        out_shape=(jax.ShapeDtypeStruct((NQ, K_NN), jnp.int32),
                   jax.ShapeDtypeStruct((NQ, K_NN * CF), jnp.float32)),
        scratch_shapes=[pltpu.VMEM((_QB, _WC), jnp.float32),
                        pltpu.VMEM((_QB, K_NN), jnp.float32),
                        pltpu.VMEM((_QB, K_NN), jnp.int32)],
    )(sq)


def _sc_gather(table, idx):
    V, D = table.shape
    B = idx.shape[0]
    info = plsc.get_sparse_core_info()
    nw = info.num_cores * info.num_subcores
    b_per_w = B // nw
    mesh = plsc.VectorSubcoreMesh(core_axis_name="c", subcore_axis_name="s")

    @functools.partial(
        pl.kernel, mesh=mesh,
        out_type=jax.ShapeDtypeStruct((B, D), jnp.float32),
        scratch_types=[
            pltpu.VMEM((b_per_w,), jnp.int32),
            pltpu.VMEM((b_per_w, D), jnp.float32),
            pltpu.SemaphoreType.DMA,
        ],
    )
    def k(table_hbm, idx_hbm, out_hbm, idx_v, rows_v, sem):
        wid = lax.axis_index("s") * info.num_cores + lax.axis_index("c")
        base = wid * b_per_w
        pltpu.sync_copy(idx_hbm.at[pl.ds(base, b_per_w)], idx_v)
        pltpu.async_copy(table_hbm.at[idx_v], rows_v, sem).wait()
        pltpu.sync_copy(rows_v, out_hbm.at[pl.ds(base, b_per_w)])

    return k(table, idx)


def _sc_gather_mul(table, idx, weights):
    V, D = table.shape
    B = idx.shape[0]
    info = plsc.get_sparse_core_info()
    nw = info.num_cores * info.num_subcores
    b_per_w = B // nw
    mesh = plsc.VectorSubcoreMesh(core_axis_name="c", subcore_axis_name="s")
    L = info.num_lanes

    b_half = b_per_w // 2
    w_flat = weights.reshape(B * CF)

    @functools.partial(
        pl.kernel, mesh=mesh,
        out_type=jax.ShapeDtypeStruct((B, D), jnp.float32),
        scratch_types=[
            pltpu.VMEM((b_half,), jnp.int32),
            pltpu.VMEM((b_half, D), jnp.float32),
            pltpu.VMEM((b_half * CF,), jnp.float32),
            pltpu.SemaphoreType.DMA,
        ],
    )
    def k(table_hbm, idx_hbm, w_hbm, out_hbm, idx_v, rows_v, w_v, sem):
        wid = lax.axis_index("s") * info.num_cores + lax.axis_index("c")
        for part in range(2):
            base = wid * b_per_w + part * b_half
            pltpu.sync_copy(idx_hbm.at[pl.ds(base, b_half)], idx_v)
            pltpu.sync_copy(w_hbm.at[pl.ds(base * CF, b_half * CF)], w_v)
            pltpu.async_copy(table_hbm.at[idx_v], rows_v, sem).wait()

            def body(i, carry):
                for h in range(CF // L):
                    rows_v[i, pl.ds(h * L, L)] = (
                        rows_v[i, pl.ds(h * L, L)] * w_v[pl.ds(i * CF + h * L, L)])
                return carry

            lax.fori_loop(0, b_half, body, 0)
            pltpu.sync_copy(rows_v, out_hbm.at[pl.ds(base, b_half)])

    return k(table, idx, w_flat)


def kernel(candidate_pts, src_keypts, tgt_pts_xyz, tgt_deep_feat_pts):
    B = src_keypts.shape[0]
    K = candidate_pts.shape[1]
    C = candidate_pts.shape[2]
    cand = candidate_pts.reshape(NQ, 3)
    tgt = tgt_pts_xyz.reshape(NT, 3)
    feat = tgt_deep_feat_pts.reshape(NT, CF)

    sqd_ball = _square_distance(cand[None], cand[None])[0]
    sq_knn = _square_distance(cand[None], tgt[None])[0]

    fps_row, g_orig = _run_fps_ball(cand, cand.T, sqd_ball)
    idx32, e_mat = _run_topk(sq_knn)

    feat_pad = jnp.pad(feat, ((0, 0), (0, 128 - CF)))
    g_perm = _sc_gather(g_orig, fps_row.reshape(NQ))[:, :3 * NSAMPLE]
    featw = _sc_gather_mul(feat_pad, idx32.reshape(NQ * K_NN),
                           e_mat.reshape(NQ * K_NN, CF))[:, :CF]

    grouped_xyz = g_perm.reshape(1, K, C, NSAMPLE, 3)
    tgt_feat_norm = featw.reshape(1, K, C, K_NN, CF)
    return jnp.concatenate([grouped_xyz, tgt_feat_norm], axis=4)
